# baseline (speedup 1.0000x reference)
import functools

import jax
import jax.numpy as jnp
from jax import lax
from jax.experimental import pallas as pl
from jax.experimental.pallas import tpu as pltpu
from jax.experimental.pallas import tpu_sc as plsc

D_MODEL = 768

_info = plsc.get_sparse_core_info()
NC, NS = _info.num_cores, _info.num_subcores
NW = NC * NS

CHUNK = 16
NBUF = 8


def _embed_sc(n_tokens: int, tokens_flat, W_E):
    b_per_w = n_tokens // NW
    n_chunks = b_per_w // CHUNK
    idx3 = tokens_flat.reshape(NW, n_chunks, CHUNK).astype(jnp.int32)
    mesh = plsc.VectorSubcoreMesh(core_axis_name="c", subcore_axis_name="s")

    @functools.partial(
        pl.kernel,
        out_type=jax.ShapeDtypeStruct((n_tokens, D_MODEL), jnp.float32),
        mesh=mesh,
        scratch_types=[
            pltpu.VMEM((n_chunks, CHUNK), jnp.int32),
            [pltpu.VMEM((CHUNK, D_MODEL), jnp.float32) for _ in range(NBUF)],
            [pltpu.SemaphoreType.DMA for _ in range(NBUF)],
            [pltpu.SemaphoreType.DMA for _ in range(NBUF)],
        ],
    )
    def k(idx_hbm, table_hbm, out_hbm, idx_v, bufs, gsems, ssems):
        wid = lax.axis_index("s") * NC + lax.axis_index("c")
        base = wid * b_per_w
        pltpu.sync_copy(idx_hbm.at[wid], idx_v)
        gathers = [None] * n_chunks
        scatters = [None] * n_chunks
        for c in range(min(NBUF, n_chunks)):
            gathers[c] = pltpu.async_copy(
                table_hbm.at[idx_v.at[c]], bufs[c], gsems[c]
            )
        for c in range(n_chunks):
            b = c % NBUF
            gathers[c].wait()
            scatters[c] = pltpu.async_copy(
                bufs[b], out_hbm.at[pl.ds(base + c * CHUNK, CHUNK)], ssems[b]
            )
            nxt = c + NBUF
            if nxt < n_chunks:
                scatters[c].wait()
                gathers[nxt] = pltpu.async_copy(
                    table_hbm.at[idx_v.at[nxt]], bufs[b], gsems[b]
                )
        for c in range(max(0, n_chunks - NBUF), n_chunks):
            scatters[c].wait()

    return k(idx3, W_E)


def kernel(tokens, W_E):
    bsz, seq = tokens.shape
    n_tokens = bsz * seq
    out = _embed_sc(n_tokens, tokens.reshape(n_tokens), W_E)
    return out.reshape(bsz, seq, D_MODEL)


# speedup vs baseline: 1.0236x; 1.0236x over previous
import functools

import jax
import jax.numpy as jnp
from jax import lax
from jax.experimental import pallas as pl
from jax.experimental.pallas import tpu as pltpu
from jax.experimental.pallas import tpu_sc as plsc

D_MODEL = 768

_info = plsc.get_sparse_core_info()
NC, NS = _info.num_cores, _info.num_subcores
NW = NC * NS

CHUNK = 32
NBUF = 5


def _embed_sc(n_tokens: int, tokens_flat, W_E):
    b_per_w = n_tokens // NW
    n_chunks = b_per_w // CHUNK
    idx3 = tokens_flat.reshape(NW, n_chunks, CHUNK).astype(jnp.int32)
    mesh = plsc.VectorSubcoreMesh(core_axis_name="c", subcore_axis_name="s")

    @functools.partial(
        pl.kernel,
        out_type=jax.ShapeDtypeStruct((n_tokens, D_MODEL), jnp.float32),
        mesh=mesh,
        scratch_types=[
            pltpu.VMEM((n_chunks, CHUNK), jnp.int32),
            [pltpu.VMEM((CHUNK, D_MODEL), jnp.float32) for _ in range(NBUF)],
            [pltpu.SemaphoreType.DMA for _ in range(NBUF)],
            [pltpu.SemaphoreType.DMA for _ in range(NBUF)],
        ],
    )
    def k(idx_hbm, table_hbm, out_hbm, idx_v, bufs, gsems, ssems):
        wid = lax.axis_index("s") * NC + lax.axis_index("c")
        base = wid * b_per_w
        pltpu.sync_copy(idx_hbm.at[wid], idx_v)
        gathers = [None] * n_chunks
        scatters = [None] * n_chunks
        for c in range(min(NBUF, n_chunks)):
            gathers[c] = pltpu.async_copy(
                table_hbm.at[idx_v.at[c]], bufs[c], gsems[c]
            )
        for c in range(n_chunks):
            b = c % NBUF
            gathers[c].wait()
            scatters[c] = pltpu.async_copy(
                bufs[b], out_hbm.at[pl.ds(base + c * CHUNK, CHUNK)], ssems[b]
            )
            nxt = c + NBUF
            if nxt < n_chunks:
                scatters[c].wait()
                gathers[nxt] = pltpu.async_copy(
                    table_hbm.at[idx_v.at[nxt]], bufs[b], gsems[b]
                )
        for c in range(max(0, n_chunks - NBUF), n_chunks):
            scatters[c].wait()

    return k(idx3, W_E)


def kernel(tokens, W_E):
    bsz, seq = tokens.shape
    n_tokens = bsz * seq
    out = _embed_sc(n_tokens, tokens.reshape(n_tokens), W_E)
    return out.reshape(bsz, seq, D_MODEL)
